# Initial kernel scaffold; baseline (speedup 1.0000x reference)
import jax
import jax.numpy as jnp
from jax.experimental import pallas as pl


def kernel(xs_targ, xs_meta, ys_meta, max_N_label):
    raise NotImplementedError("write your pallas kernel here")


# speedup vs baseline: 15.2995x; 15.2995x over previous
import functools

import jax
import jax.numpy as jnp
from jax.experimental import pallas as pl
from jax.experimental.pallas import tpu as pltpu

NUM_LABEL = 64
TQ = 512


def _proto_kernel(xm_ref, ys_ref, mask_ref, xt_ref, out_ref, protos_ref):
    j = pl.program_id(1)

    @pl.when(j == 0)
    def _build_protos():
        xm = xm_ref[0]
        ys = ys_ref[0, 0]
        n_meta = ys.shape[0]
        labels = jax.lax.broadcasted_iota(jnp.int32, (n_meta, NUM_LABEL), 1)
        onehot = (ys[:, None] == labels).astype(jnp.float32)
        sums = jax.lax.dot_general(
            onehot, xm, (((0,), (0,)), ((), ())),
            preferred_element_type=jnp.float32)
        counts = jnp.sum(onehot, axis=0)
        protos_ref[...] = sums / jnp.maximum(counts, 1.0)[:, None]

    x = xt_ref[0]
    p = protos_ref[...]
    xn = jnp.sum(x * x, axis=1)
    pn = jnp.sum(p * p, axis=1)
    xp = jax.lax.dot_general(
        x, p, (((1,), (1,)), ((), ())),
        preferred_element_type=jnp.float32)
    d2 = jnp.maximum(xn[:, None] + pn[None, :] - 2.0 * xp, 0.0)
    dist = -jnp.sqrt(d2)
    m = jnp.max(dist, axis=1, keepdims=True)
    e = jnp.exp(dist - m)
    probs = e / jnp.sum(e, axis=1, keepdims=True)
    out_ref[0] = probs * mask_ref[...]


@functools.partial(jax.jit, static_argnames=())
def kernel(xs_targ, xs_meta, ys_meta, max_N_label):
    B, N_targ, d = xs_targ.shape
    N_meta = xs_meta.shape[1]
    nq = N_targ // TQ
    ys3 = ys_meta.reshape(B, 1, N_meta).astype(jnp.int32)
    label_mask = (jnp.arange(NUM_LABEL) < max_N_label).astype(
        jnp.float32).reshape(1, NUM_LABEL)

    out = pl.pallas_call(
        _proto_kernel,
        grid=(B, nq),
        in_specs=[
            pl.BlockSpec((1, N_meta, d), lambda b, j: (b, 0, 0)),
            pl.BlockSpec((1, 1, N_meta), lambda b, j: (b, 0, 0)),
            pl.BlockSpec((1, NUM_LABEL), lambda b, j: (0, 0)),
            pl.BlockSpec((1, TQ, d), lambda b, j: (b, j, 0)),
        ],
        out_specs=pl.BlockSpec((1, TQ, NUM_LABEL), lambda b, j: (b, j, 0)),
        out_shape=jax.ShapeDtypeStruct((B, N_targ, NUM_LABEL), jnp.float32),
        scratch_shapes=[pltpu.VMEM((NUM_LABEL, d), jnp.float32)],
        compiler_params=pltpu.CompilerParams(
            dimension_semantics=("arbitrary", "arbitrary")),
    )(xs_meta, ys3, label_mask, xs_targ)
    return out.reshape(B * N_targ, NUM_LABEL)
